# baseline (speedup 1.0000x reference)
import functools

import jax
import jax.numpy as jnp
from jax.experimental import pallas as pl

_TARGET = 2048


def _norms_body(fe_ref, out_ref):
    x = fe_ref[0]
    out_ref[0, 0] = jnp.sum(x * x, axis=0)


def _norms_tc(fe):
    B, C, E = fe.shape
    return pl.pallas_call(
        _norms_body,
        grid=(B,),
        in_specs=[pl.BlockSpec((1, C, E), lambda b: (b, 0, 0))],
        out_specs=pl.BlockSpec((1, 1, E), lambda b: (b, 0, 0)),
        out_shape=jax.ShapeDtypeStruct((B, 1, E), jnp.float32),
    )(fe)


def kernel(fe):
    norms = _norms_tc(fe)[:, 0, :]
    _, idx = jax.lax.top_k(norms, _TARGET)
    idx = jnp.sort(idx, axis=-1)
    return jnp.take_along_axis(fe, idx[:, None, :], axis=2)


# speedup vs baseline: 1.4394x; 1.4394x over previous
import functools

import jax
import jax.numpy as jnp
from jax import lax
from jax.experimental import pallas as pl
from jax.experimental.pallas import tpu as pltpu
from jax.experimental.pallas import tpu_sc as plsc

_TARGET = 2048
_SHIFT = 18
_NBINS = 1 << (32 - _SHIFT)
_CAP = 8192


def _norms_body(fe_ref, out_ref):
    x = fe_ref[0]
    out_ref[0, 0] = lax.bitcast_convert_type(jnp.sum(x * x, axis=0), jnp.int32)


def _norms_tc(fe):
    B, C, E = fe.shape
    return pl.pallas_call(
        _norms_body,
        grid=(B,),
        in_specs=[pl.BlockSpec((1, C, E), lambda b: (b, 0, 0))],
        out_specs=pl.BlockSpec((1, 1, E), lambda b: (b, 0, 0)),
        out_shape=jax.ShapeDtypeStruct((B, 1, E), jnp.int32),
    )(fe)


def _make_sc(B, C, E):
    K = _TARGET
    NCH = C // 2
    mesh = plsc.VectorSubcoreMesh(core_axis_name="c", subcore_axis_name="s")

    @functools.partial(
        pl.kernel,
        out_type=jax.ShapeDtypeStruct((B, C, K), jnp.float32),
        mesh=mesh,
        compiler_params=pltpu.CompilerParams(
            needs_layout_passes=False, use_tc_tiling_on_sc=False),
        scratch_types=[
            pltpu.VMEM((E,), jnp.int32),
            pltpu.VMEM((_NBINS,), jnp.int32),
            pltpu.VMEM((_CAP + 16,), jnp.int32),
            pltpu.VMEM((_CAP + 16,), jnp.int32),
            pltpu.VMEM((K + 16,), jnp.int32),
            pltpu.VMEM((2, K), jnp.float32),
            pltpu.SemaphoreType.DMA,
            pltpu.SemaphoreType.DMA,
            pltpu.SemaphoreType.DMA,
        ],
    )
    def sc_kernel(fe_hbm, norms_hbm, out_hbm, nv, hist, cval, cidx, fidx,
                  gbuf, sem_n, sem_g, sem_o):
        ci = lax.axis_index("c")
        si = lax.axis_index("s")
        b = si
        c0 = ci * NCH

        pltpu.async_copy(norms_hbm.at[b, 0], nv, sem_n).wait()

        z16 = jnp.zeros((16,), jnp.int32)

        def zb(i, carry):
            hist[pl.ds(i * 16, 16)] = z16
            return carry

        lax.fori_loop(0, _NBINS // 16, zb, 0)

        def pa(i, vmax):
            u = nv[pl.ds(i * 16, 16)]
            bn = u >> _SHIFT
            cnt, lastm = plsc.scan_count(bn)
            plsc.addupdate_scatter(hist, [bn], cnt, mask=lastm)
            return jnp.maximum(vmax, u)

        umax_v = lax.fori_loop(0, E // 16, pa, jnp.zeros((16,), jnp.int32))
        maxbin = jnp.max(umax_v) >> _SHIFT

        def t_cond(c):
            j, cum, T, found = c
            return jnp.logical_not(found) & (j >= 0)

        def t_body(c):
            j, cum, T, found = c
            h = hist[pl.ds(j * 16, 16)]
            r = lax.rev(h, (0,))
            cs = plsc.cumsum(r) + cum
            hit = cs >= K
            nhit = plsc.all_reduce_population_count(hit)[0]
            anyhit = nhit > 0
            p = plsc.all_reduce_ffs(hit)[0]
            Tn = j * 16 + 15 - p
            return (j - 1, cum + jnp.sum(h), jnp.where(anyhit, Tn, T),
                    found | anyhit)

        _, _, T, _ = lax.while_loop(
            t_cond, t_body,
            (maxbin // 16, jnp.int32(0), jnp.int32(0), False))

        def pb(i, carry):
            pos, g = carry
            u = nv[pl.ds(i * 16, 16)]
            bn = u >> _SHIFT
            m = (bn >= T) & (pos < _CAP)
            iv = i * 16 + lax.iota(jnp.int32, 16)
            plsc.store_compressed(cval.at[pl.ds(pos, 16)], u, mask=m)
            plsc.store_compressed(cidx.at[pl.ds(pos, 16)], iv, mask=m)
            return (pos + jnp.sum(m.astype(jnp.int32)),
                    g + jnp.sum((bn > T).astype(jnp.int32)))

        M, G = lax.fori_loop(0, E // 16, pb, (jnp.int32(0), jnp.int32(0)))
        cval[pl.ds(M, 16)] = jnp.zeros((16,), jnp.int32)
        nvc = (M + 15) // 16

        lo = T << _SHIFT
        hi = lo + ((1 << _SHIFT) - 1)

        def bis(r, carry):
            lo, hi = carry
            mid = lo + ((hi - lo + 1) >> 1)

            def cb(i, acc):
                uu = cval[pl.ds(i * 16, 16)]
                return acc + (uu >= mid).astype(jnp.int32)

            cge = jnp.sum(lax.fori_loop(0, nvc, cb, z16))
            ok = cge >= K
            return jnp.where(ok, mid, lo), jnp.where(ok, hi, mid - 1)

        X, _ = lax.fori_loop(0, _SHIFT, bis, (lo, hi))

        def cg(i, acc):
            uu = cval[pl.ds(i * 16, 16)]
            return acc + (uu > X).astype(jnp.int32)

        count_gt = jnp.sum(lax.fori_loop(0, nvc, cg, z16))
        need_eq = K - count_gt

        def pc(i, carry):
            pos, eqc = carry
            uu = cval[pl.ds(i * 16, 16)]
            ii = cidx[pl.ds(i * 16, 16)]
            ok = (i * 16 + lax.iota(jnp.int32, 16)) < M
            gt = (uu > X) & ok
            eq = (uu == X) & ok
            eqi = eq.astype(jnp.int32)
            rk = plsc.cumsum(eqi) + eqc
            keep = gt | (eq & (rk <= need_eq))
            plsc.store_compressed(fidx.at[pl.ds(pos, 16)], ii, mask=keep)
            return (pos + jnp.sum(keep.astype(jnp.int32)),
                    eqc + jnp.sum(eqi))

        lax.fori_loop(0, nvc, pc, (jnp.int32(0), jnp.int32(0)))

        def src(c):
            row = (b * C + c0 + c) * E
            return fe_hbm.at[pl.ds(row, E)].at[fidx.at[pl.ds(0, K)]]

        pltpu.async_copy(src(0), gbuf.at[0], sem_g)

        def gch(c, carry):
            @pl.when(c >= 1)
            def _():
                pltpu.make_async_copy(
                    gbuf.at[(c - 1) % 2], out_hbm.at[b, c0 + c - 1],
                    sem_o).wait()

            @pl.when(c + 1 < NCH)
            def _():
                pltpu.async_copy(src(c + 1), gbuf.at[(c + 1) % 2], sem_g)

            pltpu.make_async_copy(src(c), gbuf.at[c % 2], sem_g).wait()
            pltpu.async_copy(gbuf.at[c % 2], out_hbm.at[b, c0 + c], sem_o)
            return carry

        lax.fori_loop(0, NCH, gch, 0)
        pltpu.make_async_copy(
            gbuf.at[(NCH - 1) % 2], out_hbm.at[b, c0 + NCH - 1], sem_o).wait()

    return sc_kernel


def kernel(fe):
    B, C, E = fe.shape
    norms3 = _norms_tc(fe)
    fe_flat = fe.reshape(B * C * E)
    out = _make_sc(B, C, E)(fe_flat, norms3)
    return out


# speedup vs baseline: 1.5930x; 1.1067x over previous
import functools

import jax
import jax.numpy as jnp
from jax import lax
from jax.experimental import pallas as pl
from jax.experimental.pallas import tpu as pltpu
from jax.experimental.pallas import tpu_sc as plsc

_TARGET = 2048
_SHIFT = 18
_NBINS = 1 << (32 - _SHIFT)
_CAP = 8192


def _norms_body(fe_ref, out_ref):
    x = fe_ref[0]
    out_ref[0, 0] = lax.bitcast_convert_type(jnp.sum(x * x, axis=0), jnp.int32)


def _norms_tc(fe):
    B, C, E = fe.shape
    return pl.pallas_call(
        _norms_body,
        grid=(B,),
        in_specs=[pl.BlockSpec((1, C, E), lambda b: (b, 0, 0))],
        out_specs=pl.BlockSpec((1, 1, E), lambda b: (b, 0, 0)),
        out_shape=jax.ShapeDtypeStruct((B, 1, E), jnp.int32),
    )(fe)


def _make_sc(B, C, E):
    K = _TARGET
    NCH = C // 2
    mesh = plsc.VectorSubcoreMesh(core_axis_name="c", subcore_axis_name="s")

    @functools.partial(
        pl.kernel,
        out_type=jax.ShapeDtypeStruct((B, C, K), jnp.float32),
        mesh=mesh,
        compiler_params=pltpu.CompilerParams(
            needs_layout_passes=False, use_tc_tiling_on_sc=False),
        scratch_types=[
            pltpu.VMEM((E,), jnp.int32),
            pltpu.VMEM((_NBINS,), jnp.int32),
            pltpu.VMEM((_CAP + 16,), jnp.int32),
            pltpu.VMEM((_CAP + 16,), jnp.int32),
            pltpu.VMEM((K + 16,), jnp.int32),
            pltpu.VMEM((2, K), jnp.float32),
            pltpu.SemaphoreType.DMA,
            pltpu.SemaphoreType.DMA,
            pltpu.SemaphoreType.DMA,
        ],
    )
    def sc_kernel(fe_hbm, norms_hbm, out_hbm, nv, hist, cval, cidx, fidx,
                  gbuf, sem_n, sem_g, sem_o):
        ci = lax.axis_index("c")
        si = lax.axis_index("s")
        b = si
        c0 = ci * NCH

        pltpu.async_copy(norms_hbm.at[b, 0], nv, sem_n).wait()

        z16 = jnp.zeros((16,), jnp.int32)

        def zb(i, carry):
            hist[pl.ds(i * 16, 16)] = z16
            return carry

        lax.fori_loop(0, _NBINS // 16, zb, 0)

        ones16 = jnp.ones((16,), jnp.int32)

        def pa(i, vmax):
            u0 = nv[pl.ds(i * 32, 16)]
            u1 = nv[pl.ds(i * 32 + 16, 16)]
            plsc.addupdate_scatter(hist, [u0 >> _SHIFT], ones16)
            plsc.addupdate_scatter(hist, [u1 >> _SHIFT], ones16)
            return jnp.maximum(vmax, jnp.maximum(u0, u1))

        umax_v = lax.fori_loop(0, E // 32, pa, jnp.zeros((16,), jnp.int32))
        maxbin = jnp.max(umax_v) >> _SHIFT

        def t_cond(c):
            j, cum, T, found = c
            return jnp.logical_not(found) & (j >= 0)

        def t_body(c):
            j, cum, T, found = c
            h = hist[pl.ds(j * 16, 16)]
            r = lax.rev(h, (0,))
            cs = plsc.cumsum(r) + cum
            hit = cs >= K
            nhit = plsc.all_reduce_population_count(hit)[0]
            anyhit = nhit > 0
            p = plsc.all_reduce_ffs(hit)[0]
            Tn = j * 16 + 15 - p
            return (j - 1, cum + jnp.sum(h), jnp.where(anyhit, Tn, T),
                    found | anyhit)

        _, _, T, _ = lax.while_loop(
            t_cond, t_body,
            (maxbin // 16, jnp.int32(0), jnp.int32(0), False))

        def pb(i, pos):
            u = nv[pl.ds(i * 16, 16)]
            m = ((u >> _SHIFT) >= T) & (pos < _CAP)
            iv = i * 16 + lax.iota(jnp.int32, 16)
            plsc.store_compressed(cval.at[pl.ds(pos, 16)], u, mask=m)
            plsc.store_compressed(cidx.at[pl.ds(pos, 16)], iv, mask=m)
            return pos + plsc.all_reduce_population_count(m)[0]

        M = lax.fori_loop(0, E // 16, pb, jnp.int32(0))
        cval[pl.ds(M, 16)] = jnp.zeros((16,), jnp.int32)
        nvc = (M + 15) // 16

        lo = T << _SHIFT
        hi = lo + ((1 << _SHIFT) - 1)

        def bis(r, carry):
            lo, hi = carry
            mid = lo + ((hi - lo + 1) >> 1)

            def cb(i, acc):
                uu = cval[pl.ds(i * 16, 16)]
                return acc + (uu >= mid).astype(jnp.int32)

            cge = jnp.sum(lax.fori_loop(0, nvc, cb, z16))
            ok = cge >= K
            return jnp.where(ok, mid, lo), jnp.where(ok, hi, mid - 1)

        X, _ = lax.fori_loop(0, _SHIFT, bis, (lo, hi))

        def cg(i, acc):
            uu = cval[pl.ds(i * 16, 16)]
            return acc + (uu > X).astype(jnp.int32)

        count_gt = jnp.sum(lax.fori_loop(0, nvc, cg, z16))
        need_eq = K - count_gt

        def pc(i, carry):
            pos, eqc = carry
            uu = cval[pl.ds(i * 16, 16)]
            ii = cidx[pl.ds(i * 16, 16)]
            ok = (i * 16 + lax.iota(jnp.int32, 16)) < M
            gt = (uu > X) & ok
            eq = (uu == X) & ok
            eqi = eq.astype(jnp.int32)
            rk = plsc.cumsum(eqi) + eqc
            keep = gt | (eq & (rk <= need_eq))
            plsc.store_compressed(fidx.at[pl.ds(pos, 16)], ii, mask=keep)
            return (pos + plsc.all_reduce_population_count(keep)[0],
                    eqc + plsc.all_reduce_population_count(eq)[0])

        lax.fori_loop(0, nvc, pc, (jnp.int32(0), jnp.int32(0)))

        def src(c):
            row = (b * C + c0 + c) * E
            return fe_hbm.at[pl.ds(row, E)].at[fidx.at[pl.ds(0, K)]]

        pltpu.async_copy(src(0), gbuf.at[0], sem_g)

        def gch(c, carry):
            @pl.when(c >= 1)
            def _():
                pltpu.make_async_copy(
                    gbuf.at[(c - 1) % 2], out_hbm.at[b, c0 + c - 1],
                    sem_o).wait()

            @pl.when(c + 1 < NCH)
            def _():
                pltpu.async_copy(src(c + 1), gbuf.at[(c + 1) % 2], sem_g)

            pltpu.make_async_copy(src(c), gbuf.at[c % 2], sem_g).wait()
            pltpu.async_copy(gbuf.at[c % 2], out_hbm.at[b, c0 + c], sem_o)
            return carry

        lax.fori_loop(0, NCH, gch, 0)
        pltpu.make_async_copy(
            gbuf.at[(NCH - 1) % 2], out_hbm.at[b, c0 + NCH - 1], sem_o).wait()

    return sc_kernel


def kernel(fe):
    B, C, E = fe.shape
    norms3 = _norms_tc(fe)
    fe_flat = fe.reshape(B * C * E)
    out = _make_sc(B, C, E)(fe_flat, norms3)
    return out


# speedup vs baseline: 2.1893x; 1.3743x over previous
import functools

import jax
import jax.numpy as jnp
from jax import lax
from jax.experimental import pallas as pl
from jax.experimental.pallas import tpu as pltpu
from jax.experimental.pallas import tpu_sc as plsc

_TARGET = 2048
_SHIFT = 18
_NBINS = 1 << (32 - _SHIFT)
_CAP = 8192


def _norms_body(fe_ref, out_ref):
    x = fe_ref[0]
    out_ref[0, 0] = lax.bitcast_convert_type(jnp.sum(x * x, axis=0), jnp.int32)


def _norms_tc(fe):
    B, C, E = fe.shape
    return pl.pallas_call(
        _norms_body,
        grid=(B,),
        in_specs=[pl.BlockSpec((1, C, E), lambda b: (b, 0, 0))],
        out_specs=pl.BlockSpec((1, 1, E), lambda b: (b, 0, 0)),
        out_shape=jax.ShapeDtypeStruct((B, 1, E), jnp.int32),
    )(fe)


def _make_sc(B, C, E):
    K = _TARGET
    NCH = C // 2
    mesh = plsc.VectorSubcoreMesh(core_axis_name="c", subcore_axis_name="s")

    @functools.partial(
        pl.kernel,
        out_type=jax.ShapeDtypeStruct((B, C, K), jnp.float32),
        mesh=mesh,
        compiler_params=pltpu.CompilerParams(
            needs_layout_passes=False, use_tc_tiling_on_sc=False),
        scratch_types=[
            pltpu.VMEM((E,), jnp.int32),
            pltpu.VMEM((_NBINS,), jnp.int32),
            pltpu.VMEM((_CAP + 16,), jnp.int32),
            pltpu.VMEM((_CAP + 16,), jnp.int32),
            pltpu.VMEM((K + 16,), jnp.int32),
            pltpu.VMEM((K + 16,), jnp.int32),
            pltpu.VMEM((2, K), jnp.float32),
            pltpu.SemaphoreType.DMA,
            pltpu.SemaphoreType.DMA,
            pltpu.SemaphoreType.DMA,
        ],
    )
    def sc_kernel(fe_hbm, norms_hbm, out_hbm, nv, hist, cval, cidx, fidx,
                  tidx, gbuf, sem_n, sem_g, sem_o):
        ci = lax.axis_index("c")
        si = lax.axis_index("s")
        b = si
        c0 = ci * NCH

        pltpu.async_copy(norms_hbm.at[b, 0], nv, sem_n).wait()

        z16 = jnp.zeros((16,), jnp.int32)

        def zb(i, carry):
            hist[pl.ds(i * 16, 16)] = z16
            return carry

        lax.fori_loop(0, _NBINS // 16, zb, 0)

        ones16 = jnp.ones((16,), jnp.int32)

        def pa(i, vmax):
            u0 = nv[pl.ds(i * 32, 16)]
            u1 = nv[pl.ds(i * 32 + 16, 16)]
            plsc.addupdate_scatter(hist, [u0 >> _SHIFT], ones16)
            plsc.addupdate_scatter(hist, [u1 >> _SHIFT], ones16)
            return jnp.maximum(vmax, jnp.maximum(u0, u1))

        umax_v = lax.fori_loop(0, E // 32, pa, jnp.zeros((16,), jnp.int32))
        maxbin = jnp.max(umax_v) >> _SHIFT

        def t_cond(c):
            j, cum, T, found = c
            return jnp.logical_not(found) & (j >= 0)

        def t_body(c):
            j, cum, T, found = c
            h = hist[pl.ds(j * 16, 16)]
            r = lax.rev(h, (0,))
            cs = plsc.cumsum(r) + cum
            hit = cs >= K
            nhit = plsc.all_reduce_population_count(hit)[0]
            anyhit = nhit > 0
            p = plsc.all_reduce_ffs(hit)[0]
            Tn = j * 16 + 15 - p
            return (j - 1, cum + jnp.sum(h), jnp.where(anyhit, Tn, T),
                    found | anyhit)

        _, _, T, _ = lax.while_loop(
            t_cond, t_body,
            (maxbin // 16, jnp.int32(0), jnp.int32(0), False))

        def pb(i, pos):
            u = nv[pl.ds(i * 16, 16)]
            m = ((u >> _SHIFT) >= T) & (pos < _CAP)
            iv = i * 16 + lax.iota(jnp.int32, 16)
            plsc.store_compressed(cval.at[pl.ds(pos, 16)], u, mask=m)
            plsc.store_compressed(cidx.at[pl.ds(pos, 16)], iv, mask=m)
            return pos + plsc.all_reduce_population_count(m)[0]

        M = lax.fori_loop(0, E // 16, pb, jnp.int32(0))
        cval[pl.ds(M, 16)] = jnp.zeros((16,), jnp.int32)
        nvc = (M + 15) // 16

        lo = T << _SHIFT
        hi = lo + ((1 << _SHIFT) - 1)

        def bis(r, carry):
            lo, hi = carry
            mid = lo + ((hi - lo + 1) >> 1)

            def cb(i, acc):
                uu = cval[pl.ds(i * 16, 16)]
                return acc + (uu >= mid).astype(jnp.int32)

            cge = jnp.sum(lax.fori_loop(0, nvc, cb, z16))
            ok = cge >= K
            return jnp.where(ok, mid, lo), jnp.where(ok, hi, mid - 1)

        X, _ = lax.fori_loop(0, _SHIFT, bis, (lo, hi))

        def cg(i, acc):
            uu = cval[pl.ds(i * 16, 16)]
            return acc + (uu > X).astype(jnp.int32)

        count_gt = jnp.sum(lax.fori_loop(0, nvc, cg, z16))
        need_eq = K - count_gt

        def pc(i, carry):
            pos, eqc = carry
            uu = cval[pl.ds(i * 16, 16)]
            ii = cidx[pl.ds(i * 16, 16)]
            ok = (i * 16 + lax.iota(jnp.int32, 16)) < M
            gt = (uu > X) & ok
            eq = (uu == X) & ok
            eqi = eq.astype(jnp.int32)
            rk = plsc.cumsum(eqi) + eqc
            keep = gt | (eq & (rk <= need_eq))
            plsc.store_compressed(fidx.at[pl.ds(pos, 16)], ii, mask=keep)
            return (pos + plsc.all_reduce_population_count(keep)[0],
                    eqc + plsc.all_reduce_population_count(eq)[0])

        lax.fori_loop(0, nvc, pc, (jnp.int32(0), jnp.int32(0)))

        WLEN2 = (E // 128) * 1024
        NGR = NCH // 8
        chbase = ci * NGR

        def group(cl, carry):
            def tx(i, c2):
                v = fidx[pl.ds(i * 16, 16)]
                tidx[pl.ds(i * 16, 16)] = (((v >> 7) << 10) + (v & 127)
                                           + cl * 128)
                return c2

            lax.fori_loop(0, K // 16, tx, 0)

            def src2(gi):
                base = b * (C * E) + (chbase + gi) * (8 * E)
                return fe_hbm.at[pl.ds(base, WLEN2)].at[tidx.at[pl.ds(0, K)]]

            def outdst(gi):
                return out_hbm.at[b, (chbase + gi) * 8 + cl]

            pltpu.async_copy(src2(0), gbuf.at[0], sem_g)

            def inner(gi, c2):
                @pl.when(gi >= 1)
                def _():
                    pltpu.make_async_copy(
                        gbuf.at[(gi - 1) % 2], outdst(gi - 1), sem_o).wait()

                @pl.when(gi + 1 < NGR)
                def _():
                    pltpu.async_copy(src2(gi + 1), gbuf.at[(gi + 1) % 2],
                                     sem_g)

                pltpu.make_async_copy(src2(gi), gbuf.at[gi % 2], sem_g).wait()
                pltpu.async_copy(gbuf.at[gi % 2], outdst(gi), sem_o)
                return c2

            lax.fori_loop(0, NGR, inner, 0)
            pltpu.make_async_copy(
                gbuf.at[(NGR - 1) % 2], outdst(NGR - 1), sem_o).wait()
            return carry

        lax.fori_loop(0, 8, group, 0)

    return sc_kernel


def kernel(fe):
    B, C, E = fe.shape
    norms3 = _norms_tc(fe)
    fe_perm = (fe.reshape(B, C // 8, 8, E // 128, 128)
               .transpose(0, 1, 3, 2, 4).reshape(B * C * E))
    out = _make_sc(B, C, E)(fe_perm, norms3)
    return out


# speedup vs baseline: 2.3084x; 1.0544x over previous
import functools

import jax
import jax.numpy as jnp
from jax import lax
from jax.experimental import pallas as pl
from jax.experimental.pallas import tpu as pltpu
from jax.experimental.pallas import tpu_sc as plsc

_TARGET = 2048
_SHIFT = 18
_NBINS = 1 << (32 - _SHIFT)
_CAP = 8192


def _norms_body(fe_ref, out_ref):
    x = fe_ref[0]
    out_ref[0, 0] = lax.bitcast_convert_type(jnp.sum(x * x, axis=0), jnp.int32)


def _norms_tc(fe):
    B, C, E = fe.shape
    return pl.pallas_call(
        _norms_body,
        grid=(B,),
        in_specs=[pl.BlockSpec((1, C, E), lambda b: (b, 0, 0))],
        out_specs=pl.BlockSpec((1, 1, E), lambda b: (b, 0, 0)),
        out_shape=jax.ShapeDtypeStruct((B, 1, E), jnp.int32),
    )(fe)


def _make_sc(B, C, E):
    K = _TARGET
    NCH = C // 2
    mesh = plsc.VectorSubcoreMesh(core_axis_name="c", subcore_axis_name="s")

    @functools.partial(
        pl.kernel,
        out_type=jax.ShapeDtypeStruct((B, C, K), jnp.float32),
        mesh=mesh,
        compiler_params=pltpu.CompilerParams(
            needs_layout_passes=False, use_tc_tiling_on_sc=False),
        scratch_types=[
            pltpu.VMEM((E,), jnp.int32),
            pltpu.VMEM((_NBINS,), jnp.int32),
            pltpu.VMEM((_CAP + 16,), jnp.int32),
            pltpu.VMEM((_CAP + 16,), jnp.int32),
            pltpu.VMEM((K + 16,), jnp.int32),
            pltpu.VMEM((2, K + 16), jnp.int32),
            pltpu.VMEM((2, K), jnp.float32),
            pltpu.SemaphoreType.DMA,
            pltpu.SemaphoreType.DMA,
            pltpu.SemaphoreType.DMA,
        ],
    )
    def sc_kernel(fe_hbm, norms_hbm, out_hbm, nv, hist, cval, cidx, fidx,
                  tidx, gbuf, sem_n, sem_g, sem_o):
        ci = lax.axis_index("c")
        si = lax.axis_index("s")
        b = si
        c0 = ci * NCH

        pltpu.async_copy(norms_hbm.at[b, 0], nv, sem_n).wait()

        z16 = jnp.zeros((16,), jnp.int32)

        def zb(i, carry):
            hist[pl.ds(i * 32, 16)] = z16
            hist[pl.ds(i * 32 + 16, 16)] = z16
            return carry

        lax.fori_loop(0, _NBINS // 32, zb, 0)

        ones16 = jnp.ones((16,), jnp.int32)

        def pa(i, vmax):
            u0 = nv[pl.ds(i * 32, 16)]
            u1 = nv[pl.ds(i * 32 + 16, 16)]
            plsc.addupdate_scatter(hist, [u0 >> _SHIFT], ones16)
            plsc.addupdate_scatter(hist, [u1 >> _SHIFT], ones16)
            return jnp.maximum(vmax, jnp.maximum(u0, u1))

        umax_v = lax.fori_loop(0, E // 32, pa, jnp.zeros((16,), jnp.int32))
        maxbin = jnp.max(umax_v) >> _SHIFT

        def t_cond(c):
            j, cum, T, found = c
            return jnp.logical_not(found) & (j >= 0)

        def t_body(c):
            j, cum, T, found = c
            h = hist[pl.ds(j * 16, 16)]
            r = lax.rev(h, (0,))
            cs = plsc.cumsum(r) + cum
            hit = cs >= K
            nhit = plsc.all_reduce_population_count(hit)[0]
            anyhit = nhit > 0
            p = plsc.all_reduce_ffs(hit)[0]
            Tn = j * 16 + 15 - p
            return (j - 1, cum + jnp.sum(h), jnp.where(anyhit, Tn, T),
                    found | anyhit)

        _, _, T, _ = lax.while_loop(
            t_cond, t_body,
            (maxbin // 16, jnp.int32(0), jnp.int32(0), False))

        iot = lax.iota(jnp.int32, 16)

        def pb(i, pos):
            u0 = nv[pl.ds(i * 32, 16)]
            u1 = nv[pl.ds(i * 32 + 16, 16)]
            m0 = ((u0 >> _SHIFT) >= T) & (pos < _CAP)
            plsc.store_compressed(cval.at[pl.ds(pos, 16)], u0, mask=m0)
            plsc.store_compressed(cidx.at[pl.ds(pos, 16)], i * 32 + iot,
                                  mask=m0)
            pos1 = pos + plsc.all_reduce_population_count(m0)[0]
            m1 = ((u1 >> _SHIFT) >= T) & (pos1 < _CAP)
            plsc.store_compressed(cval.at[pl.ds(pos1, 16)], u1, mask=m1)
            plsc.store_compressed(cidx.at[pl.ds(pos1, 16)], i * 32 + 16 + iot,
                                  mask=m1)
            return pos1 + plsc.all_reduce_population_count(m1)[0]

        M = lax.fori_loop(0, E // 32, pb, jnp.int32(0))
        cval[pl.ds(M, 16)] = jnp.zeros((16,), jnp.int32)
        nvc = (M + 15) // 16

        lo = T << _SHIFT
        hi = lo + ((1 << _SHIFT) - 1)

        def bis(r, carry):
            lo, hi = carry
            mid = lo + ((hi - lo + 1) >> 1)

            def cb(i, acc):
                uu = cval[pl.ds(i * 16, 16)]
                return acc + (uu >= mid).astype(jnp.int32)

            cge = jnp.sum(lax.fori_loop(0, nvc, cb, z16))
            ok = cge >= K
            return jnp.where(ok, mid, lo), jnp.where(ok, hi, mid - 1)

        X, _ = lax.fori_loop(0, _SHIFT, bis, (lo, hi))

        def cg(i, acc):
            uu = cval[pl.ds(i * 16, 16)]
            return acc + (uu > X).astype(jnp.int32)

        count_gt = jnp.sum(lax.fori_loop(0, nvc, cg, z16))
        need_eq = K - count_gt

        def pc(i, carry):
            pos, eqc = carry
            uu = cval[pl.ds(i * 16, 16)]
            ii = cidx[pl.ds(i * 16, 16)]
            ok = (i * 16 + lax.iota(jnp.int32, 16)) < M
            gt = (uu > X) & ok
            eq = (uu == X) & ok
            eqi = eq.astype(jnp.int32)
            rk = plsc.cumsum(eqi) + eqc
            keep = gt | (eq & (rk <= need_eq))
            plsc.store_compressed(fidx.at[pl.ds(pos, 16)], ii, mask=keep)
            return (pos + plsc.all_reduce_population_count(keep)[0],
                    eqc + plsc.all_reduce_population_count(eq)[0])

        lax.fori_loop(0, nvc, pc, (jnp.int32(0), jnp.int32(0)))

        WLEN2 = (E // 128) * 1024
        NGR = NCH // 8
        chbase = ci * NGR

        def build(cl, buf):
            def tx(i, c2):
                v = fidx[pl.ds(i * 16, 16)]
                tidx[buf, pl.ds(i * 16, 16)] = (((v >> 7) << 10) + (v & 127)
                                                + cl * 128)
                return c2

            lax.fori_loop(0, K // 16, tx, 0)

        build(jnp.int32(0), jnp.int32(0) % 2)

        def group(cl, carry):
            def src2(gi):
                base = b * (C * E) + (chbase + gi) * (8 * E)
                return fe_hbm.at[pl.ds(base, WLEN2)].at[
                    tidx.at[cl % 2].at[pl.ds(0, K)]]

            def outdst(gi):
                return out_hbm.at[b, (chbase + gi) * 8 + cl]

            pltpu.async_copy(src2(0), gbuf.at[0], sem_g)

            @pl.when(cl + 1 < 8)
            def _():
                build(cl + 1, (cl + 1) % 2)

            def inner(gi, c2):
                @pl.when(gi >= 1)
                def _():
                    pltpu.make_async_copy(
                        gbuf.at[(gi - 1) % 2], outdst(gi - 1), sem_o).wait()

                @pl.when(gi + 1 < NGR)
                def _():
                    pltpu.async_copy(src2(gi + 1), gbuf.at[(gi + 1) % 2],
                                     sem_g)

                pltpu.make_async_copy(src2(gi), gbuf.at[gi % 2], sem_g).wait()
                pltpu.async_copy(gbuf.at[gi % 2], outdst(gi), sem_o)
                return c2

            lax.fori_loop(0, NGR, inner, 0)
            pltpu.make_async_copy(
                gbuf.at[(NGR - 1) % 2], outdst(NGR - 1), sem_o).wait()
            return carry

        lax.fori_loop(0, 8, group, 0)

    return sc_kernel


def kernel(fe):
    B, C, E = fe.shape
    norms3 = _norms_tc(fe)
    fe_perm = (fe.reshape(B, C // 8, 8, E // 128, 128)
               .transpose(0, 1, 3, 2, 4).reshape(B * C * E))
    out = _make_sc(B, C, E)(fe_perm, norms3)
    return out


# speedup vs baseline: 2.4419x; 1.0578x over previous
import functools

import jax
import jax.numpy as jnp
from jax import lax
from jax.experimental import pallas as pl
from jax.experimental.pallas import tpu as pltpu
from jax.experimental.pallas import tpu_sc as plsc

_TARGET = 2048
_SHIFT = 18
_NBINS = 1 << (32 - _SHIFT)
_CAP = 8192


def _norms_body(fe_ref, out_ref):
    x = fe_ref[0]
    out_ref[0, 0] = lax.bitcast_convert_type(jnp.sum(x * x, axis=0), jnp.int32)


def _norms_tc(fe):
    B, C, E = fe.shape
    return pl.pallas_call(
        _norms_body,
        grid=(B,),
        in_specs=[pl.BlockSpec((1, C, E), lambda b: (b, 0, 0))],
        out_specs=pl.BlockSpec((1, 1, E), lambda b: (b, 0, 0)),
        out_shape=jax.ShapeDtypeStruct((B, 1, E), jnp.int32),
    )(fe)


def _make_sc(B, C, E):
    K = _TARGET
    NCH = C // 2
    mesh = plsc.VectorSubcoreMesh(core_axis_name="c", subcore_axis_name="s")

    @functools.partial(
        pl.kernel,
        out_type=jax.ShapeDtypeStruct((B, C * K), jnp.float32),
        mesh=mesh,
        compiler_params=pltpu.CompilerParams(
            needs_layout_passes=False, use_tc_tiling_on_sc=False),
        scratch_types=[
            pltpu.VMEM((E,), jnp.int32),
            pltpu.VMEM((_NBINS,), jnp.int32),
            pltpu.VMEM((_CAP + 16,), jnp.int32),
            pltpu.VMEM((_CAP + 16,), jnp.int32),
            pltpu.VMEM((K + 16,), jnp.int32),
            pltpu.VMEM((8 * K + 16,), jnp.int32),
            pltpu.VMEM((2, 8 * K), jnp.float32),
            pltpu.SemaphoreType.DMA,
            pltpu.SemaphoreType.DMA,
            pltpu.SemaphoreType.DMA,
        ],
    )
    def sc_kernel(fe_hbm, norms_hbm, out_hbm, nv, hist, cval, cidx, fidx,
                  tidx, gbuf, sem_n, sem_g, sem_o):
        ci = lax.axis_index("c")
        si = lax.axis_index("s")
        b = si
        c0 = ci * NCH

        pltpu.async_copy(norms_hbm.at[b, 0], nv, sem_n).wait()

        z16 = jnp.zeros((16,), jnp.int32)

        def zb(i, carry):
            hist[pl.ds(i * 32, 16)] = z16
            hist[pl.ds(i * 32 + 16, 16)] = z16
            return carry

        lax.fori_loop(0, _NBINS // 32, zb, 0)

        ones16 = jnp.ones((16,), jnp.int32)

        def pa(i, vmax):
            u0 = nv[pl.ds(i * 32, 16)]
            u1 = nv[pl.ds(i * 32 + 16, 16)]
            plsc.addupdate_scatter(hist, [u0 >> _SHIFT], ones16)
            plsc.addupdate_scatter(hist, [u1 >> _SHIFT], ones16)
            return jnp.maximum(vmax, jnp.maximum(u0, u1))

        umax_v = lax.fori_loop(0, E // 32, pa, jnp.zeros((16,), jnp.int32))
        maxbin = jnp.max(umax_v) >> _SHIFT

        def t_cond(c):
            j, cum, T, found = c
            return jnp.logical_not(found) & (j >= 0)

        def t_body(c):
            j, cum, T, found = c
            h = hist[pl.ds(j * 16, 16)]
            r = lax.rev(h, (0,))
            cs = plsc.cumsum(r) + cum
            hit = cs >= K
            nhit = plsc.all_reduce_population_count(hit)[0]
            anyhit = nhit > 0
            p = plsc.all_reduce_ffs(hit)[0]
            Tn = j * 16 + 15 - p
            return (j - 1, cum + jnp.sum(h), jnp.where(anyhit, Tn, T),
                    found | anyhit)

        _, _, T, _ = lax.while_loop(
            t_cond, t_body,
            (maxbin // 16, jnp.int32(0), jnp.int32(0), False))

        iot = lax.iota(jnp.int32, 16)

        def pb(i, pos):
            u0 = nv[pl.ds(i * 32, 16)]
            u1 = nv[pl.ds(i * 32 + 16, 16)]
            m0 = ((u0 >> _SHIFT) >= T) & (pos < _CAP)
            plsc.store_compressed(cval.at[pl.ds(pos, 16)], u0, mask=m0)
            plsc.store_compressed(cidx.at[pl.ds(pos, 16)], i * 32 + iot,
                                  mask=m0)
            pos1 = pos + plsc.all_reduce_population_count(m0)[0]
            m1 = ((u1 >> _SHIFT) >= T) & (pos1 < _CAP)
            plsc.store_compressed(cval.at[pl.ds(pos1, 16)], u1, mask=m1)
            plsc.store_compressed(cidx.at[pl.ds(pos1, 16)], i * 32 + 16 + iot,
                                  mask=m1)
            return pos1 + plsc.all_reduce_population_count(m1)[0]

        M = lax.fori_loop(0, E // 32, pb, jnp.int32(0))
        cval[pl.ds(M, 16)] = jnp.zeros((16,), jnp.int32)
        nvc = (M + 15) // 16

        lo = T << _SHIFT
        hi = lo + ((1 << _SHIFT) - 1)

        def bis(r, carry):
            lo, hi = carry
            mid = lo + ((hi - lo + 1) >> 1)

            def cb(i, acc):
                uu = cval[pl.ds(i * 16, 16)]
                return acc + (uu >= mid).astype(jnp.int32)

            cge = jnp.sum(lax.fori_loop(0, nvc, cb, z16))
            ok = cge >= K
            return jnp.where(ok, mid, lo), jnp.where(ok, hi, mid - 1)

        X, _ = lax.fori_loop(0, _SHIFT, bis, (lo, hi))

        def cg(i, acc):
            uu = cval[pl.ds(i * 16, 16)]
            return acc + (uu > X).astype(jnp.int32)

        count_gt = jnp.sum(lax.fori_loop(0, nvc, cg, z16))
        need_eq = K - count_gt

        def pc(i, carry):
            pos, eqc = carry
            uu = cval[pl.ds(i * 16, 16)]
            ii = cidx[pl.ds(i * 16, 16)]
            ok = (i * 16 + lax.iota(jnp.int32, 16)) < M
            gt = (uu > X) & ok
            eq = (uu == X) & ok
            eqi = eq.astype(jnp.int32)
            rk = plsc.cumsum(eqi) + eqc
            keep = gt | (eq & (rk <= need_eq))
            plsc.store_compressed(fidx.at[pl.ds(pos, 16)], ii, mask=keep)
            return (pos + plsc.all_reduce_population_count(keep)[0],
                    eqc + plsc.all_reduce_population_count(eq)[0])

        lax.fori_loop(0, nvc, pc, (jnp.int32(0), jnp.int32(0)))

        WLEN2 = (E // 128) * 1024
        NGR = NCH // 8
        chbase = ci * NGR

        def txc(i, c2):
            v = fidx[pl.ds(i * 16, 16)]
            t0 = ((v >> 7) << 10) + (v & 127)
            for cl in range(8):
                tidx[pl.ds(cl * K + i * 16, 16)] = t0 + cl * 128
            return c2

        lax.fori_loop(0, K // 16, txc, 0)

        def src3(w):
            base = b * (C * E) + (chbase + w) * (8 * E)
            return fe_hbm.at[pl.ds(base, WLEN2)].at[tidx.at[pl.ds(0, 8 * K)]]

        def outdst3(w):
            return out_hbm.at[b, pl.ds((chbase + w) * (8 * K), 8 * K)]

        pltpu.async_copy(src3(0), gbuf.at[0], sem_g)

        def wloop(w, c2):
            @pl.when(w >= 1)
            def _():
                pltpu.make_async_copy(
                    gbuf.at[(w - 1) % 2], outdst3(w - 1), sem_o).wait()

            @pl.when(w + 1 < NGR)
            def _():
                pltpu.async_copy(src3(w + 1), gbuf.at[(w + 1) % 2], sem_g)

            pltpu.make_async_copy(src3(w), gbuf.at[w % 2], sem_g).wait()
            pltpu.async_copy(gbuf.at[w % 2], outdst3(w), sem_o)
            return c2

        lax.fori_loop(0, NGR, wloop, 0)
        pltpu.make_async_copy(
            gbuf.at[(NGR - 1) % 2], outdst3(NGR - 1), sem_o).wait()

    return sc_kernel


def kernel(fe):
    B, C, E = fe.shape
    norms3 = _norms_tc(fe)
    fe_perm = (fe.reshape(B, C // 8, 8, E // 128, 128)
               .transpose(0, 1, 3, 2, 4).reshape(B * C * E))
    out = _make_sc(B, C, E)(fe_perm, norms3)
    return out.reshape(B, C, _TARGET)


# speedup vs baseline: 2.5823x; 1.0575x over previous
import functools

import jax
import jax.numpy as jnp
from jax import lax
from jax.experimental import pallas as pl
from jax.experimental.pallas import tpu as pltpu
from jax.experimental.pallas import tpu_sc as plsc

_TARGET = 2048
_SHIFT = 18
_NBINS = 1 << (32 - _SHIFT)
_CAP = 8192


def _norms_body(fe_ref, out_ref):
    x = fe_ref[0]
    out_ref[0, 0] = lax.bitcast_convert_type(jnp.sum(x * x, axis=0), jnp.int32)


def _norms_tc(fe):
    B, C, E = fe.shape
    return pl.pallas_call(
        _norms_body,
        grid=(B,),
        in_specs=[pl.BlockSpec((1, C, E), lambda b: (b, 0, 0))],
        out_specs=pl.BlockSpec((1, 1, E), lambda b: (b, 0, 0)),
        out_shape=jax.ShapeDtypeStruct((B, 1, E), jnp.int32),
    )(fe)


def _make_sc(B, C, E):
    K = _TARGET
    NCH = C // 2
    mesh = plsc.VectorSubcoreMesh(core_axis_name="c", subcore_axis_name="s")

    @functools.partial(
        pl.kernel,
        out_type=jax.ShapeDtypeStruct((B, C * K), jnp.float32),
        mesh=mesh,
        compiler_params=pltpu.CompilerParams(
            needs_layout_passes=False, use_tc_tiling_on_sc=False),
        scratch_types=[
            pltpu.VMEM((E,), jnp.int32),
            pltpu.VMEM((_NBINS,), jnp.int32),
            pltpu.VMEM((_CAP + 64,), jnp.int32),
            pltpu.VMEM((_CAP + 16,), jnp.int32),
            pltpu.VMEM((K + 16,), jnp.int32),
            pltpu.VMEM((8 * K + 16,), jnp.int32),
            pltpu.VMEM((2, 8 * K), jnp.float32),
            pltpu.SemaphoreType.DMA,
            pltpu.SemaphoreType.DMA,
            pltpu.SemaphoreType.DMA,
        ],
    )
    def sc_kernel(fe_hbm, norms_hbm, out_hbm, nv, hist, cval, cidx, fidx,
                  tidx, gbuf, sem_n, sem_g, sem_o):
        ci = lax.axis_index("c")
        si = lax.axis_index("s")
        b = si
        c0 = ci * NCH

        pltpu.async_copy(norms_hbm.at[b, 0], nv, sem_n).wait()

        z16 = jnp.zeros((16,), jnp.int32)

        def zb(i, carry):
            hist[pl.ds(i * 32, 16)] = z16
            hist[pl.ds(i * 32 + 16, 16)] = z16
            return carry

        lax.fori_loop(0, _NBINS // 32, zb, 0)

        ones16 = jnp.ones((16,), jnp.int32)

        def pa(i, vmax):
            u0 = nv[pl.ds(i * 32, 16)]
            u1 = nv[pl.ds(i * 32 + 16, 16)]
            plsc.addupdate_scatter(hist, [u0 >> _SHIFT], ones16)
            plsc.addupdate_scatter(hist, [u1 >> _SHIFT], ones16)
            return jnp.maximum(vmax, jnp.maximum(u0, u1))

        umax_v = lax.fori_loop(0, E // 32, pa, jnp.zeros((16,), jnp.int32))
        maxbin = jnp.max(umax_v) >> _SHIFT

        def t_cond(c):
            j, cum, T, found = c
            return jnp.logical_not(found) & (j >= 0)

        def t_body(c):
            j, cum, T, found = c
            h = hist[pl.ds(j * 16, 16)]
            r = lax.rev(h, (0,))
            cs = plsc.cumsum(r) + cum
            hit = cs >= K
            nhit = plsc.all_reduce_population_count(hit)[0]
            anyhit = nhit > 0
            p = plsc.all_reduce_ffs(hit)[0]
            Tn = j * 16 + 15 - p
            return (j - 1, cum + jnp.sum(h), jnp.where(anyhit, Tn, T),
                    found | anyhit)

        _, _, T, _ = lax.while_loop(
            t_cond, t_body,
            (maxbin // 16, jnp.int32(0), jnp.int32(0), False))

        iot = lax.iota(jnp.int32, 16)

        def pb(i, pos):
            u0 = nv[pl.ds(i * 32, 16)]
            u1 = nv[pl.ds(i * 32 + 16, 16)]
            m0 = ((u0 >> _SHIFT) >= T) & (pos < _CAP)
            plsc.store_compressed(cval.at[pl.ds(pos, 16)], u0, mask=m0)
            plsc.store_compressed(cidx.at[pl.ds(pos, 16)], i * 32 + iot,
                                  mask=m0)
            pos1 = pos + plsc.all_reduce_population_count(m0)[0]
            m1 = ((u1 >> _SHIFT) >= T) & (pos1 < _CAP)
            plsc.store_compressed(cval.at[pl.ds(pos1, 16)], u1, mask=m1)
            plsc.store_compressed(cidx.at[pl.ds(pos1, 16)], i * 32 + 16 + iot,
                                  mask=m1)
            return pos1 + plsc.all_reduce_population_count(m1)[0]

        M = lax.fori_loop(0, E // 32, pb, jnp.int32(0))
        for q in range(4):
            cval[pl.ds(M + q * 16, 16)] = z16
        nvc = (M + 15) // 16
        nvc4 = (M + 63) // 64

        lo = T << _SHIFT
        hi = lo + ((1 << _SHIFT) - 1)

        def bis(r, carry):
            lo, hi = carry
            mid = lo + ((hi - lo + 1) >> 1)

            def cb(i, acc):
                a0 = (cval[pl.ds(i * 64, 16)] >= mid).astype(jnp.int32)
                a1 = (cval[pl.ds(i * 64 + 16, 16)] >= mid).astype(jnp.int32)
                a2 = (cval[pl.ds(i * 64 + 32, 16)] >= mid).astype(jnp.int32)
                a3 = (cval[pl.ds(i * 64 + 48, 16)] >= mid).astype(jnp.int32)
                return acc + ((a0 + a1) + (a2 + a3))

            cge = jnp.sum(lax.fori_loop(0, nvc4, cb, z16))
            ok = cge >= K
            return jnp.where(ok, mid, lo), jnp.where(ok, hi, mid - 1)

        X, _ = lax.fori_loop(0, _SHIFT, bis, (lo, hi))

        def cg(i, acc):
            a0 = (cval[pl.ds(i * 64, 16)] > X).astype(jnp.int32)
            a1 = (cval[pl.ds(i * 64 + 16, 16)] > X).astype(jnp.int32)
            a2 = (cval[pl.ds(i * 64 + 32, 16)] > X).astype(jnp.int32)
            a3 = (cval[pl.ds(i * 64 + 48, 16)] > X).astype(jnp.int32)
            return acc + ((a0 + a1) + (a2 + a3))

        count_gt = jnp.sum(lax.fori_loop(0, nvc4, cg, z16))
        need_eq = K - count_gt

        def pc(i, carry):
            pos, eqc = carry
            uu = cval[pl.ds(i * 16, 16)]
            ii = cidx[pl.ds(i * 16, 16)]
            ok = (i * 16 + lax.iota(jnp.int32, 16)) < M
            gt = (uu > X) & ok
            eq = (uu == X) & ok
            eqi = eq.astype(jnp.int32)
            rk = plsc.cumsum(eqi) + eqc
            keep = gt | (eq & (rk <= need_eq))
            plsc.store_compressed(fidx.at[pl.ds(pos, 16)], ii, mask=keep)
            return (pos + plsc.all_reduce_population_count(keep)[0],
                    eqc + plsc.all_reduce_population_count(eq)[0])

        lax.fori_loop(0, nvc, pc, (jnp.int32(0), jnp.int32(0)))

        WLEN2 = (E // 128) * 1024
        NGR = NCH // 8
        chbase = ci * NGR

        def txc(i, c2):
            v = fidx[pl.ds(i * 16, 16)]
            t0 = ((v >> 7) << 10) + (v & 127)
            for cl in range(8):
                tidx[pl.ds(cl * K + i * 16, 16)] = t0 + cl * 128
            return c2

        lax.fori_loop(0, K // 16, txc, 0)

        def src3(w):
            base = b * (C * E) + (chbase + w) * (8 * E)
            return fe_hbm.at[pl.ds(base, WLEN2)].at[tidx.at[pl.ds(0, 8 * K)]]

        def outdst3(w):
            return out_hbm.at[b, pl.ds((chbase + w) * (8 * K), 8 * K)]

        pltpu.async_copy(src3(0), gbuf.at[0], sem_g)

        def wloop(w, c2):
            @pl.when(w >= 1)
            def _():
                pltpu.make_async_copy(
                    gbuf.at[(w - 1) % 2], outdst3(w - 1), sem_o).wait()

            @pl.when(w + 1 < NGR)
            def _():
                pltpu.async_copy(src3(w + 1), gbuf.at[(w + 1) % 2], sem_g)

            pltpu.make_async_copy(src3(w), gbuf.at[w % 2], sem_g).wait()
            pltpu.async_copy(gbuf.at[w % 2], outdst3(w), sem_o)
            return c2

        lax.fori_loop(0, NGR, wloop, 0)
        pltpu.make_async_copy(
            gbuf.at[(NGR - 1) % 2], outdst3(NGR - 1), sem_o).wait()

    return sc_kernel


def kernel(fe):
    B, C, E = fe.shape
    norms3 = _norms_tc(fe)
    fe_perm = (fe.reshape(B, C // 8, 8, E // 128, 128)
               .transpose(0, 1, 3, 2, 4).reshape(B * C * E))
    out = _make_sc(B, C, E)(fe_perm, norms3)
    return out.reshape(B, C, _TARGET)


# speedup vs baseline: 2.6280x; 1.0177x over previous
import functools

import jax
import jax.numpy as jnp
from jax import lax
from jax.experimental import pallas as pl
from jax.experimental.pallas import tpu as pltpu
from jax.experimental.pallas import tpu_sc as plsc

_TARGET = 2048
_SHIFT = 18
_NBINS = 1 << (32 - _SHIFT)
_CAP = 8192


def _norms_body(fe_ref, out_ref):
    x = fe_ref[0]
    out_ref[0, 0] = lax.bitcast_convert_type(jnp.sum(x * x, axis=0), jnp.int32)


def _norms_tc(fe):
    B, C, E = fe.shape
    return pl.pallas_call(
        _norms_body,
        grid=(B,),
        in_specs=[pl.BlockSpec((1, C, E), lambda b: (b, 0, 0))],
        out_specs=pl.BlockSpec((1, 1, E), lambda b: (b, 0, 0)),
        out_shape=jax.ShapeDtypeStruct((B, 1, E), jnp.int32),
    )(fe)


def _make_sc(B, C, E):
    K = _TARGET
    NCH = C // 2
    mesh = plsc.VectorSubcoreMesh(core_axis_name="c", subcore_axis_name="s")

    @functools.partial(
        pl.kernel,
        out_type=jax.ShapeDtypeStruct((B, C * K), jnp.float32),
        mesh=mesh,
        compiler_params=pltpu.CompilerParams(
            needs_layout_passes=False, use_tc_tiling_on_sc=False),
        scratch_types=[
            pltpu.VMEM((E,), jnp.int32),
            pltpu.VMEM((_NBINS,), jnp.int32),
            pltpu.VMEM((_CAP + 64,), jnp.int32),
            pltpu.VMEM((_CAP + 16,), jnp.int32),
            pltpu.VMEM((K + 16,), jnp.int32),
            pltpu.VMEM((8 * K + 16,), jnp.int32),
            pltpu.VMEM((2, 8 * K), jnp.float32),
            pltpu.SemaphoreType.DMA,
            pltpu.SemaphoreType.DMA,
            pltpu.SemaphoreType.DMA,
        ],
    )
    def sc_kernel(fe_hbm, norms_hbm, out_hbm, nv, hist, cval, cidx, fidx,
                  tidx, gbuf, sem_n, sem_g, sem_o):
        ci = lax.axis_index("c")
        si = lax.axis_index("s")
        b = si
        c0 = ci * NCH

        pltpu.async_copy(norms_hbm.at[b, 0], nv, sem_n).wait()

        z16 = jnp.zeros((16,), jnp.int32)

        def zb(i, carry):
            hist[pl.ds(i * 32, 16)] = z16
            hist[pl.ds(i * 32 + 16, 16)] = z16
            return carry

        lax.fori_loop(0, _NBINS // 32, zb, 0)

        ones16 = jnp.ones((16,), jnp.int32)

        def pa(i, vmax):
            u0 = nv[pl.ds(i * 64, 16)]
            u1 = nv[pl.ds(i * 64 + 16, 16)]
            u2 = nv[pl.ds(i * 64 + 32, 16)]
            u3 = nv[pl.ds(i * 64 + 48, 16)]
            plsc.addupdate_scatter(hist, [u0 >> _SHIFT], ones16)
            plsc.addupdate_scatter(hist, [u1 >> _SHIFT], ones16)
            plsc.addupdate_scatter(hist, [u2 >> _SHIFT], ones16)
            plsc.addupdate_scatter(hist, [u3 >> _SHIFT], ones16)
            return jnp.maximum(vmax, jnp.maximum(jnp.maximum(u0, u1),
                                                 jnp.maximum(u2, u3)))

        umax_v = lax.fori_loop(0, E // 64, pa, jnp.zeros((16,), jnp.int32))
        maxbin = jnp.max(umax_v) >> _SHIFT

        def t_cond(c):
            j, cum, T, found = c
            return jnp.logical_not(found) & (j >= 0)

        def t_body(c):
            j, cum, T, found = c
            h = hist[pl.ds(j * 16, 16)]
            r = lax.rev(h, (0,))
            cs = plsc.cumsum(r) + cum
            hit = cs >= K
            nhit = plsc.all_reduce_population_count(hit)[0]
            anyhit = nhit > 0
            p = plsc.all_reduce_ffs(hit)[0]
            Tn = j * 16 + 15 - p
            return (j - 1, cum + jnp.sum(h), jnp.where(anyhit, Tn, T),
                    found | anyhit)

        _, _, T, _ = lax.while_loop(
            t_cond, t_body,
            (maxbin // 16, jnp.int32(0), jnp.int32(0), False))

        iot = lax.iota(jnp.int32, 16)

        def pb(i, pos):
            u0 = nv[pl.ds(i * 32, 16)]
            u1 = nv[pl.ds(i * 32 + 16, 16)]
            m0 = ((u0 >> _SHIFT) >= T) & (pos < _CAP)
            plsc.store_compressed(cval.at[pl.ds(pos, 16)], u0, mask=m0)
            plsc.store_compressed(cidx.at[pl.ds(pos, 16)], i * 32 + iot,
                                  mask=m0)
            pos1 = pos + plsc.all_reduce_population_count(m0)[0]
            m1 = ((u1 >> _SHIFT) >= T) & (pos1 < _CAP)
            plsc.store_compressed(cval.at[pl.ds(pos1, 16)], u1, mask=m1)
            plsc.store_compressed(cidx.at[pl.ds(pos1, 16)], i * 32 + 16 + iot,
                                  mask=m1)
            return pos1 + plsc.all_reduce_population_count(m1)[0]

        M = lax.fori_loop(0, E // 32, pb, jnp.int32(0))
        for q in range(4):
            cval[pl.ds(M + q * 16, 16)] = z16
        nvc = (M + 15) // 16
        nvc4 = (M + 63) // 64

        lo = T << _SHIFT
        hi = lo + ((1 << _SHIFT) - 1)

        def bis(r, carry):
            lo, hi = carry
            mid = lo + ((hi - lo + 1) >> 1)

            def cb(i, acc):
                a0 = (cval[pl.ds(i * 64, 16)] >= mid).astype(jnp.int32)
                a1 = (cval[pl.ds(i * 64 + 16, 16)] >= mid).astype(jnp.int32)
                a2 = (cval[pl.ds(i * 64 + 32, 16)] >= mid).astype(jnp.int32)
                a3 = (cval[pl.ds(i * 64 + 48, 16)] >= mid).astype(jnp.int32)
                return acc + ((a0 + a1) + (a2 + a3))

            cge = jnp.sum(lax.fori_loop(0, nvc4, cb, z16))
            ok = cge >= K
            return jnp.where(ok, mid, lo), jnp.where(ok, hi, mid - 1)

        X, _ = lax.fori_loop(0, _SHIFT, bis, (lo, hi))

        def cg(i, acc):
            a0 = (cval[pl.ds(i * 64, 16)] > X).astype(jnp.int32)
            a1 = (cval[pl.ds(i * 64 + 16, 16)] > X).astype(jnp.int32)
            a2 = (cval[pl.ds(i * 64 + 32, 16)] > X).astype(jnp.int32)
            a3 = (cval[pl.ds(i * 64 + 48, 16)] > X).astype(jnp.int32)
            return acc + ((a0 + a1) + (a2 + a3))

        count_gt = jnp.sum(lax.fori_loop(0, nvc4, cg, z16))
        need_eq = K - count_gt

        def pc(i, carry):
            pos, eqc = carry
            uu = cval[pl.ds(i * 16, 16)]
            ii = cidx[pl.ds(i * 16, 16)]
            ok = (i * 16 + lax.iota(jnp.int32, 16)) < M
            gt = (uu > X) & ok
            eq = (uu == X) & ok
            eqi = eq.astype(jnp.int32)
            rk = plsc.cumsum(eqi) + eqc
            keep = gt | (eq & (rk <= need_eq))
            plsc.store_compressed(fidx.at[pl.ds(pos, 16)], ii, mask=keep)
            return (pos + plsc.all_reduce_population_count(keep)[0],
                    eqc + plsc.all_reduce_population_count(eq)[0])

        lax.fori_loop(0, nvc, pc, (jnp.int32(0), jnp.int32(0)))

        WLEN2 = (E // 128) * 1024
        NGR = NCH // 8
        chbase = ci * NGR

        def txc(i, c2):
            v = fidx[pl.ds(i * 16, 16)]
            t0 = ((v >> 7) << 10) + (v & 127)
            for cl in range(8):
                tidx[pl.ds(cl * K + i * 16, 16)] = t0 + cl * 128
            return c2

        lax.fori_loop(0, K // 16, txc, 0)

        def src3(w):
            base = b * (C * E) + (chbase + w) * (8 * E)
            return fe_hbm.at[pl.ds(base, WLEN2)].at[tidx.at[pl.ds(0, 8 * K)]]

        def outdst3(w):
            return out_hbm.at[b, pl.ds((chbase + w) * (8 * K), 8 * K)]

        pltpu.async_copy(src3(0), gbuf.at[0], sem_g)

        def wloop(w, c2):
            @pl.when(w >= 1)
            def _():
                pltpu.make_async_copy(
                    gbuf.at[(w - 1) % 2], outdst3(w - 1), sem_o).wait()

            @pl.when(w + 1 < NGR)
            def _():
                pltpu.async_copy(src3(w + 1), gbuf.at[(w + 1) % 2], sem_g)

            pltpu.make_async_copy(src3(w), gbuf.at[w % 2], sem_g).wait()
            pltpu.async_copy(gbuf.at[w % 2], outdst3(w), sem_o)
            return c2

        lax.fori_loop(0, NGR, wloop, 0)
        pltpu.make_async_copy(
            gbuf.at[(NGR - 1) % 2], outdst3(NGR - 1), sem_o).wait()

    return sc_kernel


def kernel(fe):
    B, C, E = fe.shape
    norms3 = _norms_tc(fe)
    fe_perm = (fe.reshape(B, C // 8, 8, E // 128, 128)
               .transpose(0, 1, 3, 2, 4).reshape(B * C * E))
    out = _make_sc(B, C, E)(fe_perm, norms3)
    return out.reshape(B, C, _TARGET)
